# Initial kernel scaffold; baseline (speedup 1.0000x reference)
import jax
import jax.numpy as jnp
from jax.experimental import pallas as pl


def kernel(queries, keys):
    raise NotImplementedError("write your pallas kernel here")


# speedup vs baseline: 1.7567x; 1.7567x over previous
import functools

import jax
import jax.numpy as jnp
from jax.experimental import pallas as pl
from jax.experimental.pallas import tpu as pltpu

_TOP_K = 32
_NUM_KEYS = 100000
_PAD_W = 100352
_QB = 64
_NUM_Q = 1024


def _topk_body(q_ref, kt_ref, vals_ref, idx_ref, s_ref):
    q = q_ref[...]
    qn = q / (jnp.sqrt(jnp.sum(q * q, axis=1, keepdims=True)) + 1e-8)
    kt = kt_ref[...]
    knorm = jnp.sqrt(jnp.sum(kt * kt, axis=0, keepdims=True)) + 1e-8
    ktn = kt / knorm
    s = jax.lax.dot_general(
        qn.astype(jnp.bfloat16),
        ktn.astype(jnp.bfloat16),
        (((1,), (0,)), ((), ())),
        preferred_element_type=jnp.float32,
    )
    lane = jax.lax.broadcasted_iota(jnp.int32, (_QB, _PAD_W), 1)
    s = jnp.where(lane < _NUM_KEYS, s, -jnp.inf)
    s_ref[...] = s

    col = jax.lax.broadcasted_iota(jnp.int32, (_QB, _TOP_K), 1)

    def body(i, carry):
        vals, idxs = carry
        sv = s_ref[...]
        m = jnp.max(sv, axis=1)
        am = jnp.min(jnp.where(sv == m[:, None], lane, _PAD_W), axis=1)
        s_ref[...] = jnp.where(lane == am[:, None], -jnp.inf, sv)
        vals = jnp.where(col == i, m[:, None], vals)
        idxs = jnp.where(col == i, am[:, None], idxs)
        return vals, idxs

    vals0 = jnp.zeros((_QB, _TOP_K), jnp.float32)
    idxs0 = jnp.zeros((_QB, _TOP_K), jnp.int32)
    vals, idxs = jax.lax.fori_loop(0, _TOP_K, body, (vals0, idxs0))
    vals_ref[...] = vals
    idx_ref[...] = idxs


@jax.jit
def kernel(queries, keys):
    kt = jnp.pad(keys, ((0, _PAD_W - _NUM_KEYS), (0, 0))).T
    grid = (_NUM_Q // _QB,)
    vals, idxs = pl.pallas_call(
        _topk_body,
        grid=grid,
        in_specs=[
            pl.BlockSpec((_QB, 16), lambda i: (i, 0)),
            pl.BlockSpec((16, _PAD_W), lambda i: (0, 0)),
        ],
        out_specs=[
            pl.BlockSpec((_QB, _TOP_K), lambda i: (i, 0)),
            pl.BlockSpec((_QB, _TOP_K), lambda i: (i, 0)),
        ],
        out_shape=[
            jax.ShapeDtypeStruct((_NUM_Q, _TOP_K), jnp.float32),
            jax.ShapeDtypeStruct((_NUM_Q, _TOP_K), jnp.int32),
        ],
        scratch_shapes=[pltpu.VMEM((_QB, _PAD_W), jnp.float32)],
    )(queries, kt)
    return vals, idxs


# speedup vs baseline: 2.2470x; 1.2791x over previous
import functools

import jax
import jax.numpy as jnp
from jax.experimental import pallas as pl
from jax.experimental.pallas import tpu as pltpu
from jax.experimental.pallas import tpu_sc as plsc

_TOP_K = 32
_NUM_KEYS = 100000
_PAD_W = 100352
_G = 128
_NUM_GROUPS = _PAD_W // _G
_GPAD = 896
_QB = 64
_NUM_Q = 1024
_NCAND = _TOP_K * _G
_QB3 = 16
_KPO = 8
_OPG = _G // _KPO
_NOCT = _NUM_Q * _TOP_K * _OPG
_SC_CORES = 2
_SC_SUBCORES = 16
_SC_WORKERS = _SC_CORES * _SC_SUBCORES
_B_PER_W = _NOCT // _SC_WORKERS
_CHUNK = 512
_NCHUNK = _B_PER_W // _CHUNK


def _knorm_body(k_ref, kn_ref):
    k = k_ref[...]
    n = jnp.sqrt(jnp.sum(k * k, axis=1, keepdims=True)) + 1e-8
    kn_ref[...] = k / n


def _k0_normalize(keys_pad):
    return pl.pallas_call(
        _knorm_body,
        grid=(8,),
        in_specs=[pl.BlockSpec((_PAD_W // 8, 16), lambda i: (i, 0))],
        out_specs=pl.BlockSpec((_PAD_W // 8, 16), lambda i: (i, 0)),
        out_shape=jax.ShapeDtypeStruct((_PAD_W, 16), jnp.float32),
    )(keys_pad)


def _groups_body(q_ref, ktn_ref, gwin_ref, s_ref, r_ref):
    q = q_ref[...]
    qn = q / (jnp.sqrt(jnp.sum(q * q, axis=1, keepdims=True)) + 1e-8)
    s = jax.lax.dot_general(
        qn.astype(jnp.bfloat16),
        ktn_ref[...].astype(jnp.bfloat16),
        (((1,), (0,)), ((), ())),
        preferred_element_type=jnp.float32,
    )
    lane = jax.lax.broadcasted_iota(jnp.int32, (_QB, _PAD_W), 1)
    s = jnp.where(lane < _NUM_KEYS, s, -jnp.inf)
    s_ref[...] = s
    r = jnp.max(s_ref[...].reshape(_QB, _NUM_GROUPS, _G), axis=2)
    glane = jax.lax.broadcasted_iota(jnp.int32, (_QB, _GPAD), 1)
    r_ref[...] = jnp.where(
        glane < _NUM_GROUPS,
        jnp.pad(r, ((0, 0), (0, _GPAD - _NUM_GROUPS))),
        -jnp.inf,
    )

    col = jax.lax.broadcasted_iota(jnp.int32, (_QB, _TOP_K), 1)

    def body(i, gwin):
        r = r_ref[...]
        m = jnp.max(r, axis=1)
        g = jnp.min(jnp.where(r == m[:, None], glane, _GPAD), axis=1)
        r_ref[...] = jnp.where(glane == g[:, None], -jnp.inf, r)
        return jnp.where(col == i, g[:, None], gwin)

    gwin = jax.lax.fori_loop(
        0, _TOP_K, body, jnp.zeros((_QB, _TOP_K), jnp.int32)
    )
    gwin_ref[...] = gwin


def _k1_groups(queries, ktn):
    return pl.pallas_call(
        _groups_body,
        grid=(_NUM_Q // _QB,),
        in_specs=[
            pl.BlockSpec((_QB, 16), lambda i: (i, 0)),
            pl.BlockSpec((16, _PAD_W), lambda i: (0, 0)),
        ],
        out_specs=pl.BlockSpec((_QB, _TOP_K), lambda i: (i, 0)),
        out_shape=jax.ShapeDtypeStruct((_NUM_Q, _TOP_K), jnp.int32),
        scratch_shapes=[
            pltpu.VMEM((_QB, _PAD_W), jnp.float32),
            pltpu.VMEM((_QB, _GPAD), jnp.float32),
        ],
    )(queries, ktn)


def _gather_body(table_ref, idx_ref, out_ref, idx_v, rows_v, sem):
    wid = jax.lax.axis_index("s") * _SC_CORES + jax.lax.axis_index("c")
    w_base = wid * _B_PER_W

    @pl.loop(0, _NCHUNK)
    def _chunk(c):
        base = w_base + c * _CHUNK
        pltpu.sync_copy(idx_ref.at[pl.ds(base, _CHUNK)], idx_v)
        pltpu.async_copy(table_ref.at[idx_v], rows_v, sem).wait()
        pltpu.sync_copy(rows_v, out_ref.at[pl.ds(base, _CHUNK)])


def _k2_gather(kn_packed, oct_ids):
    mesh = plsc.VectorSubcoreMesh(core_axis_name="c", subcore_axis_name="s")
    k2 = functools.partial(
        pl.kernel,
        out_type=jax.ShapeDtypeStruct((_NOCT, _KPO * 16), jnp.float32),
        mesh=mesh,
        scratch_types=[
            pltpu.VMEM((_CHUNK,), jnp.int32),
            pltpu.VMEM((_CHUNK, _KPO * 16), jnp.float32),
            pltpu.SemaphoreType.DMA,
        ],
    )(_gather_body)
    return k2(kn_packed, oct_ids)


def _final_body(q_ref, gkt_ref, cid_ref, vals_ref, idx_ref, s_ref):
    q = q_ref[...]
    qn = q / (jnp.sqrt(jnp.sum(q * q, axis=1, keepdims=True)) + 1e-8)
    sall = jax.lax.dot_general(
        qn.astype(jnp.bfloat16),
        gkt_ref[...].astype(jnp.bfloat16),
        (((1,), (0,)), ((), ())),
        preferred_element_type=jnp.float32,
    )
    s3 = sall.reshape(_QB3, _QB3, _NCAND)
    qi = jax.lax.broadcasted_iota(jnp.int32, (_QB3, _QB3, _NCAND), 0)
    ji = jax.lax.broadcasted_iota(jnp.int32, (_QB3, _QB3, _NCAND), 1)
    s_ref[...] = jnp.sum(jnp.where(qi == ji, s3, 0.0), axis=1)

    cid = cid_ref[...]
    col = jax.lax.broadcasted_iota(jnp.int32, (_QB3, _TOP_K), 1)
    big = jnp.int32(1 << 30)

    def body(i, carry):
        vals, idxs = carry
        s = s_ref[...]
        m = jnp.max(s, axis=1)
        wid_ = jnp.min(jnp.where(s == m[:, None], cid, big), axis=1)
        s_ref[...] = jnp.where(cid == wid_[:, None], -jnp.inf, s)
        vals = jnp.where(col == i, m[:, None], vals)
        idxs = jnp.where(col == i, wid_[:, None], idxs)
        return vals, idxs

    vals, idxs = jax.lax.fori_loop(
        0,
        _TOP_K,
        body,
        (
            jnp.zeros((_QB3, _TOP_K), jnp.float32),
            jnp.zeros((_QB3, _TOP_K), jnp.int32),
        ),
    )
    vals_ref[...] = vals
    idx_ref[...] = idxs


def _k3_final(queries, gkt, cand_ids2):
    return pl.pallas_call(
        _final_body,
        grid=(_NUM_Q // _QB3,),
        in_specs=[
            pl.BlockSpec((_QB3, 16), lambda i: (i, 0)),
            pl.BlockSpec((16, _QB3 * _NCAND), lambda i: (0, i)),
            pl.BlockSpec((_QB3, _NCAND), lambda i: (i, 0)),
        ],
        out_specs=[
            pl.BlockSpec((_QB3, _TOP_K), lambda i: (i, 0)),
            pl.BlockSpec((_QB3, _TOP_K), lambda i: (i, 0)),
        ],
        out_shape=[
            jax.ShapeDtypeStruct((_NUM_Q, _TOP_K), jnp.float32),
            jax.ShapeDtypeStruct((_NUM_Q, _TOP_K), jnp.int32),
        ],
        scratch_shapes=[pltpu.VMEM((_QB3, _NCAND), jnp.float32)],
    )(queries, gkt, cand_ids2)


@jax.jit
def kernel(queries, keys):
    keys_pad = jnp.pad(keys, ((0, _PAD_W - _NUM_KEYS), (0, 0)))
    kn = _k0_normalize(keys_pad)
    ktn = kn.T
    gwin = _k1_groups(queries, ktn)
    cand_ids2 = (
        gwin[:, :, None] * _G + jnp.arange(_G, dtype=jnp.int32)[None, None, :]
    ).reshape(_NUM_Q, _NCAND)
    oct_ids = (
        gwin[:, :, None] * _OPG
        + jnp.arange(_OPG, dtype=jnp.int32)[None, None, :]
    ).reshape(_NOCT)
    kn_packed = kn.reshape(_PAD_W // _KPO, _KPO * 16)
    gk = _k2_gather(kn_packed, oct_ids)
    gkt = gk.reshape(_NUM_Q * _NCAND, 16).T
    vals, idxs = _k3_final(queries, gkt, cand_ids2)
    return vals, idxs


# speedup vs baseline: 7.3342x; 3.2640x over previous
import functools

import jax
import jax.numpy as jnp
from jax.experimental import pallas as pl
from jax.experimental.pallas import tpu as pltpu
from jax.experimental.pallas import tpu_sc as plsc

_TOP_K = 32
_NUM_KEYS = 100000
_PAD_W = 100352
_G = 128
_NUM_GROUPS = _PAD_W // _G
_GPAD = 896
_QB = 32
_NUM_Q = 1024
_NCAND = _TOP_K * _G
_QB3 = 32
_NROW = _NUM_Q * _TOP_K
_SC_CORES = 2
_SC_SUBCORES = 16
_SC_WORKERS = _SC_CORES * _SC_SUBCORES
_B_PER_W = _NROW // _SC_WORKERS
_CHUNK = 512
_NCHUNK = _B_PER_W // _CHUNK


def _knorm_body(k_ref, kn_ref):
    k = k_ref[...]
    n = jnp.sqrt(jnp.sum(k * k, axis=1, keepdims=True)) + 1e-8
    kn_ref[...] = k / n


def _k0_normalize(keys_pad):
    return pl.pallas_call(
        _knorm_body,
        grid=(8,),
        in_specs=[pl.BlockSpec((_PAD_W // 8, 16), lambda i: (i, 0))],
        out_specs=pl.BlockSpec((_PAD_W // 8, 16), lambda i: (i, 0)),
        out_shape=jax.ShapeDtypeStruct((_PAD_W, 16), jnp.float32),
    )(keys_pad)


def _groups_body(q_ref, ktn_ref, gwin_ref, s_ref, r_ref):
    q = q_ref[...]
    qn = q / (jnp.sqrt(jnp.sum(q * q, axis=1, keepdims=True)) + 1e-8)
    s = jax.lax.dot_general(
        qn.astype(jnp.bfloat16),
        ktn_ref[...].astype(jnp.bfloat16),
        (((1,), (0,)), ((), ())),
        preferred_element_type=jnp.float32,
    )
    lane = jax.lax.broadcasted_iota(jnp.int32, (_QB, _PAD_W), 1)
    s = jnp.where(lane < _NUM_KEYS, s, -jnp.inf)
    s_ref[...] = s.reshape(_QB, _NUM_GROUPS, _G)
    r = jnp.max(s_ref[...], axis=2)
    glane = jax.lax.broadcasted_iota(jnp.int32, (_QB, _GPAD), 1)
    r_ref[...] = jnp.where(
        glane < _NUM_GROUPS,
        jnp.pad(r, ((0, 0), (0, _GPAD - _NUM_GROUPS))),
        -jnp.inf,
    )

    col = jax.lax.broadcasted_iota(jnp.int32, (_QB, _TOP_K), 1)

    def body(i, gwin):
        r = r_ref[...]
        m = jnp.max(r, axis=1)
        g = jnp.min(jnp.where(r == m[:, None], glane, _GPAD), axis=1)
        r_ref[...] = jnp.where(glane == g[:, None], -jnp.inf, r)
        return jnp.where(col == i, g[:, None], gwin)

    gwin = jax.lax.fori_loop(
        0, _TOP_K, body, jnp.zeros((_QB, _TOP_K), jnp.int32)
    )
    gwin_ref[...] = gwin


def _k1_groups(queries, ktn):
    return pl.pallas_call(
        _groups_body,
        grid=(_NUM_Q // _QB,),
        in_specs=[
            pl.BlockSpec((_QB, 16), lambda i: (i, 0)),
            pl.BlockSpec((16, _PAD_W), lambda i: (0, 0)),
        ],
        out_specs=[
            pl.BlockSpec((_QB, _TOP_K), lambda i: (i, 0)),
            pl.BlockSpec((_QB, _NUM_GROUPS, _G), lambda i: (i, 0, 0)),
        ],
        out_shape=[
            jax.ShapeDtypeStruct((_NUM_Q, _TOP_K), jnp.int32),
            jax.ShapeDtypeStruct((_NUM_Q, _NUM_GROUPS, _G), jnp.float32),
        ],
        scratch_shapes=[
            pltpu.VMEM((_QB, _GPAD), jnp.float32),
        ],
    )(queries, ktn)


def _gather_body(table_ref, idx_ref, out_ref, idx_v, rows_v, sem):
    wid = jax.lax.axis_index("s") * _SC_CORES + jax.lax.axis_index("c")
    w_base = wid * _B_PER_W

    @pl.loop(0, _NCHUNK)
    def _chunk(c):
        base = w_base + c * _CHUNK
        pltpu.sync_copy(idx_ref.at[pl.ds(base, _CHUNK)], idx_v)
        pltpu.async_copy(table_ref.at[idx_v], rows_v, sem).wait()
        pltpu.sync_copy(rows_v, out_ref.at[pl.ds(base, _CHUNK)])


def _k2_gather(sim_table, row_ids):
    mesh = plsc.VectorSubcoreMesh(core_axis_name="c", subcore_axis_name="s")
    k2 = functools.partial(
        pl.kernel,
        out_type=jax.ShapeDtypeStruct((_NROW, _G), jnp.float32),
        mesh=mesh,
        scratch_types=[
            pltpu.VMEM((_CHUNK,), jnp.int32),
            pltpu.VMEM((_CHUNK, _G), jnp.float32),
            pltpu.SemaphoreType.DMA,
        ],
    )(_gather_body)
    return k2(sim_table, row_ids)


def _final_body(sc_ref, cid_ref, vals_ref, idx_ref, s_ref):
    s_ref[...] = sc_ref[...]
    cid = cid_ref[...]
    col = jax.lax.broadcasted_iota(jnp.int32, (_QB3, _TOP_K), 1)
    big = jnp.int32(1 << 30)

    def body(i, carry):
        vals, idxs = carry
        s = s_ref[...]
        m = jnp.max(s, axis=1)
        wid_ = jnp.min(jnp.where(s == m[:, None], cid, big), axis=1)
        s_ref[...] = jnp.where(cid == wid_[:, None], -jnp.inf, s)
        vals = jnp.where(col == i, m[:, None], vals)
        idxs = jnp.where(col == i, wid_[:, None], idxs)
        return vals, idxs

    vals, idxs = jax.lax.fori_loop(
        0,
        _TOP_K,
        body,
        (
            jnp.zeros((_QB3, _TOP_K), jnp.float32),
            jnp.zeros((_QB3, _TOP_K), jnp.int32),
        ),
    )
    vals_ref[...] = vals
    idx_ref[...] = idxs


def _k3_final(s_cand, cand_ids2):
    return pl.pallas_call(
        _final_body,
        grid=(_NUM_Q // _QB3,),
        in_specs=[
            pl.BlockSpec((_QB3, _NCAND), lambda i: (i, 0)),
            pl.BlockSpec((_QB3, _NCAND), lambda i: (i, 0)),
        ],
        out_specs=[
            pl.BlockSpec((_QB3, _TOP_K), lambda i: (i, 0)),
            pl.BlockSpec((_QB3, _TOP_K), lambda i: (i, 0)),
        ],
        out_shape=[
            jax.ShapeDtypeStruct((_NUM_Q, _TOP_K), jnp.float32),
            jax.ShapeDtypeStruct((_NUM_Q, _TOP_K), jnp.int32),
        ],
        scratch_shapes=[pltpu.VMEM((_QB3, _NCAND), jnp.float32)],
    )(s_cand, cand_ids2)


@jax.jit
def kernel(queries, keys):
    keys_pad = jnp.pad(keys, ((0, _PAD_W - _NUM_KEYS), (0, 0)))
    kn = _k0_normalize(keys_pad)
    ktn = kn.T
    gwin, s3d = _k1_groups(queries, ktn)
    sim_table = s3d.reshape(_NUM_Q * _NUM_GROUPS, _G)
    qid = jnp.arange(_NUM_Q, dtype=jnp.int32)[:, None]
    row_ids = (qid * _NUM_GROUPS + gwin).reshape(_NROW)
    gs = _k2_gather(sim_table, row_ids)
    s_cand = gs.reshape(_NUM_Q, _NCAND)
    cand_ids2 = (
        gwin[:, :, None] * _G + jnp.arange(_G, dtype=jnp.int32)[None, None, :]
    ).reshape(_NUM_Q, _NCAND)
    vals, idxs = _k3_final(s_cand, cand_ids2)
    return vals, idxs


# speedup vs baseline: 13.0824x; 1.7837x over previous
import functools

import jax
import jax.numpy as jnp
from jax.experimental import pallas as pl
from jax.experimental.pallas import tpu as pltpu
from jax.experimental.pallas import tpu_sc as plsc

_TOP_K = 32
_NUM_KEYS = 100000
_PAD_W = 100352
_G = 128
_NUM_GROUPS = _PAD_W // _G
_GPAD = 896
_QB = 32
_NUM_Q = 1024
_NCAND = _TOP_K * _G
_QB3 = 256
_NROW = _NUM_Q * _TOP_K
_SC_CORES = 2
_SC_SUBCORES = 16
_SC_WORKERS = _SC_CORES * _SC_SUBCORES
_B_PER_W = _NROW // _SC_WORKERS
_CHUNK = 512
_NCHUNK = _B_PER_W // _CHUNK


def _knorm_body(k_ref, kn_ref):
    k = k_ref[...]
    n = jnp.sqrt(jnp.sum(k * k, axis=1, keepdims=True)) + 1e-8
    kn_ref[...] = k / n


def _k0_normalize(keys_pad):
    return pl.pallas_call(
        _knorm_body,
        grid=(8,),
        in_specs=[pl.BlockSpec((_PAD_W // 8, 16), lambda i: (i, 0))],
        out_specs=pl.BlockSpec((_PAD_W // 8, 16), lambda i: (i, 0)),
        out_shape=jax.ShapeDtypeStruct((_PAD_W, 16), jnp.float32),
    )(keys_pad)


def _groups_body(q_ref, ktn_ref, r_ref, s_ref):
    q = q_ref[...]
    qn = q / (jnp.sqrt(jnp.sum(q * q, axis=1, keepdims=True)) + 1e-8)
    s = jax.lax.dot_general(
        qn.astype(jnp.bfloat16),
        ktn_ref[...].astype(jnp.bfloat16),
        (((1,), (0,)), ((), ())),
        preferred_element_type=jnp.float32,
    )
    lane = jax.lax.broadcasted_iota(jnp.int32, (_QB, _PAD_W), 1)
    s = jnp.where(lane < _NUM_KEYS, s, -jnp.inf)
    s_ref[...] = s.reshape(_QB, _NUM_GROUPS, _G)
    r = jnp.max(s_ref[...], axis=2)
    glane = jax.lax.broadcasted_iota(jnp.int32, (_QB, _GPAD), 1)
    r_ref[...] = jnp.where(
        glane < _NUM_GROUPS,
        jnp.pad(r, ((0, 0), (0, _GPAD - _NUM_GROUPS))),
        -jnp.inf,
    )


def _k1_groups(queries, ktn):
    return pl.pallas_call(
        _groups_body,
        grid=(_NUM_Q // _QB,),
        in_specs=[
            pl.BlockSpec((_QB, 16), lambda i: (i, 0)),
            pl.BlockSpec((16, _PAD_W), lambda i: (0, 0)),
        ],
        out_specs=[
            pl.BlockSpec((_QB, _GPAD), lambda i: (i, 0)),
            pl.BlockSpec((_QB, _NUM_GROUPS, _G), lambda i: (i, 0, 0)),
        ],
        out_shape=[
            jax.ShapeDtypeStruct((_NUM_Q, _GPAD), jnp.float32),
            jax.ShapeDtypeStruct((_NUM_Q, _NUM_GROUPS, _G), jnp.float32),
        ],
    )(queries, ktn)


def _phaseb_body(r_in_ref, gwin_ref, r_ref):
    r_ref[...] = r_in_ref[...]
    glane = jax.lax.broadcasted_iota(jnp.int32, (_NUM_Q, _GPAD), 1)
    col = jax.lax.broadcasted_iota(jnp.int32, (_NUM_Q, _TOP_K), 1)

    def body(i, gwin):
        r = r_ref[...]
        m = jnp.max(r, axis=1)
        g = jnp.min(jnp.where(r == m[:, None], glane, _GPAD), axis=1)
        r_ref[...] = jnp.where(glane == g[:, None], -jnp.inf, r)
        return jnp.where(col == i, g[:, None], gwin)

    gwin_ref[...] = jax.lax.fori_loop(
        0, _TOP_K, body, jnp.zeros((_NUM_Q, _TOP_K), jnp.int32)
    )


def _k1b_select_groups(r_full):
    return pl.pallas_call(
        _phaseb_body,
        out_shape=jax.ShapeDtypeStruct((_NUM_Q, _TOP_K), jnp.int32),
        scratch_shapes=[pltpu.VMEM((_NUM_Q, _GPAD), jnp.float32)],
    )(r_full)


def _gather_body(table_ref, idx_ref, out_ref, idx_v, rows_v, sem):
    wid = jax.lax.axis_index("s") * _SC_CORES + jax.lax.axis_index("c")
    w_base = wid * _B_PER_W

    @pl.loop(0, _NCHUNK)
    def _chunk(c):
        base = w_base + c * _CHUNK
        pltpu.sync_copy(idx_ref.at[pl.ds(base, _CHUNK)], idx_v)
        pltpu.async_copy(table_ref.at[idx_v], rows_v, sem).wait()
        pltpu.sync_copy(rows_v, out_ref.at[pl.ds(base, _CHUNK)])


def _k2_gather(sim_table, row_ids):
    mesh = plsc.VectorSubcoreMesh(core_axis_name="c", subcore_axis_name="s")
    k2 = functools.partial(
        pl.kernel,
        out_type=jax.ShapeDtypeStruct((_NROW, _G), jnp.float32),
        mesh=mesh,
        scratch_types=[
            pltpu.VMEM((_CHUNK,), jnp.int32),
            pltpu.VMEM((_CHUNK, _G), jnp.float32),
            pltpu.SemaphoreType.DMA,
        ],
    )(_gather_body)
    return k2(sim_table, row_ids)


def _final_body(sc_ref, cid_ref, vals_ref, idx_ref, s_ref):
    s_ref[...] = sc_ref[...]
    cid = cid_ref[...]
    col = jax.lax.broadcasted_iota(jnp.int32, (_QB3, _TOP_K), 1)
    big = jnp.int32(1 << 30)

    def body(i, carry):
        vals, idxs = carry
        s = s_ref[...]
        m = jnp.max(s, axis=1)
        wid_ = jnp.min(jnp.where(s == m[:, None], cid, big), axis=1)
        s_ref[...] = jnp.where(cid == wid_[:, None], -jnp.inf, s)
        vals = jnp.where(col == i, m[:, None], vals)
        idxs = jnp.where(col == i, wid_[:, None], idxs)
        return vals, idxs

    vals, idxs = jax.lax.fori_loop(
        0,
        _TOP_K,
        body,
        (
            jnp.zeros((_QB3, _TOP_K), jnp.float32),
            jnp.zeros((_QB3, _TOP_K), jnp.int32),
        ),
    )
    vals_ref[...] = vals
    idx_ref[...] = idxs


def _k3_final(s_cand, cand_ids2):
    return pl.pallas_call(
        _final_body,
        grid=(_NUM_Q // _QB3,),
        in_specs=[
            pl.BlockSpec((_QB3, _NCAND), lambda i: (i, 0)),
            pl.BlockSpec((_QB3, _NCAND), lambda i: (i, 0)),
        ],
        out_specs=[
            pl.BlockSpec((_QB3, _TOP_K), lambda i: (i, 0)),
            pl.BlockSpec((_QB3, _TOP_K), lambda i: (i, 0)),
        ],
        out_shape=[
            jax.ShapeDtypeStruct((_NUM_Q, _TOP_K), jnp.float32),
            jax.ShapeDtypeStruct((_NUM_Q, _TOP_K), jnp.int32),
        ],
        scratch_shapes=[pltpu.VMEM((_QB3, _NCAND), jnp.float32)],
    )(s_cand, cand_ids2)


@jax.jit
def kernel(queries, keys):
    keys_pad = jnp.pad(keys, ((0, _PAD_W - _NUM_KEYS), (0, 0)))
    kn = _k0_normalize(keys_pad)
    ktn = kn.T
    r_full, s3d = _k1_groups(queries, ktn)
    gwin = _k1b_select_groups(r_full)
    sim_table = s3d.reshape(_NUM_Q * _NUM_GROUPS, _G)
    qid = jnp.arange(_NUM_Q, dtype=jnp.int32)[:, None]
    row_ids = (qid * _NUM_GROUPS + gwin).reshape(_NROW)
    gs = _k2_gather(sim_table, row_ids)
    s_cand = gs.reshape(_NUM_Q, _NCAND)
    cand_ids2 = (
        gwin[:, :, None] * _G + jnp.arange(_G, dtype=jnp.int32)[None, None, :]
    ).reshape(_NUM_Q, _NCAND)
    vals, idxs = _k3_final(s_cand, cand_ids2)
    return vals, idxs
